# baseline (speedup 1.0000x reference)
import functools

import jax
import jax.numpy as jnp
from jax import lax
from jax.experimental import pallas as pl
from jax.experimental.pallas import tpu as pltpu
from jax.experimental.pallas import tpu_sc as plsc

B = 16384
D = 64
AUDIO_SCALE = 1.0
METADATA_SCALE = 1.0

_info = plsc.get_sparse_core_info()
NC, NS = _info.num_cores, _info.num_subcores
NW = NC * NS
BPW = B // NW
NQ = BPW // 128
CH = 256
NCHK = BPW // CH
QC = CH // 128

CB = 2048


def _tp_project(tableT, w):
    n = tableT.shape[1]
    grid = (n + CB - 1) // CB

    def body(t_ref, w_ref, o_ref):
        o_ref[...] = lax.dot_general(
            t_ref[...], w_ref[...], (((0,), (0,)), ((), ())),
            preferred_element_type=jnp.float32)

    return pl.pallas_call(
        body,
        grid=(grid,),
        in_specs=[
            pl.BlockSpec((D, CB), lambda i: (0, i)),
            pl.BlockSpec((D, D), lambda i: (0, 0)),
        ],
        out_specs=pl.BlockSpec((CB, D), lambda i: (i, 0)),
        out_shape=jax.ShapeDtypeStruct((n, D), jnp.float32),
    )(tableT, w)


def _sc_gather(nodes3, nodesh3, audio2, aid_tab, bid_tab, art2, alb2):
    mesh = plsc.VectorSubcoreMesh(core_axis_name="c", subcore_axis_name="s")

    @functools.partial(
        pl.kernel,
        mesh=mesh,
        out_type=(
            jax.ShapeDtypeStruct((B, 128), jnp.float32),
            jax.ShapeDtypeStruct((B, 128), jnp.float32),
            jax.ShapeDtypeStruct((B, 128), jnp.float32),
            jax.ShapeDtypeStruct((NW, NQ, 128), jnp.int32),
            jax.ShapeDtypeStruct((NW, NQ, 128), jnp.int32),
        ),
        scratch_types=[
            pltpu.VMEM((NQ, 128), jnp.int32),
            pltpu.VMEM((NQ, 128), jnp.int32),
            pltpu.VMEM((NQ, 128), jnp.int32),
            pltpu.VMEM((NQ, 128), jnp.int32),
            pltpu.VMEM((NQ, 128), jnp.int32),
            pltpu.VMEM((NQ, 128), jnp.int32),
            pltpu.VMEM((CH, 128), jnp.float32),
            pltpu.VMEM((CH, 128), jnp.float32),
            pltpu.VMEM((CH, 128), jnp.float32),
            pltpu.SemaphoreType.DMA,
            pltpu.SemaphoreType.DMA,
            pltpu.SemaphoreType.DMA,
        ],
    )
    def k(nodes_hbm, nodesh_hbm, audio_hbm, aid_hbm, bid_hbm, atab_hbm, btab_hbm,
          audio_out, art_out, alb_out, aid_out, bid_out,
          idx_v, idxh_v, aid_v, bid_v, aid2_v, bid2_v,
          audio_v, art_v, alb_v, sem_ids, sem_audio, sem_tab):
        wid = lax.axis_index("s") * NC + lax.axis_index("c")
        pltpu.sync_copy(nodes_hbm.at[wid], idx_v)
        pltpu.sync_copy(nodesh_hbm.at[wid], idxh_v)
        for h in range(NCHK):
            cbase = wid * BPW + h * CH
            audio_cps = []
            id_cps = []
            for qc in range(QC):
                q = h * QC + qc
                audio_cps.append(pltpu.async_copy(
                    audio_hbm.at[idxh_v.at[q]],
                    audio_v.at[pl.ds(qc * 128, 128)], sem_audio))
                id_cps.append(pltpu.async_copy(
                    aid_hbm.at[idx_v.at[q]], aid_v.at[q], sem_ids))
                id_cps.append(pltpu.async_copy(
                    bid_hbm.at[idx_v.at[q]], bid_v.at[q], sem_ids))
            for c in id_cps:
                c.wait()
            for qc in range(QC):
                q = h * QC + qc
                for j in range(8):
                    s = pl.ds(j * 16, 16)
                    aid2_v[q, s] = lax.shift_right_logical(aid_v[q, s], 1)
                    bid2_v[q, s] = lax.shift_right_logical(bid_v[q, s], 1)
            tab_cps = []
            for qc in range(QC):
                q = h * QC + qc
                tab_cps.append(pltpu.async_copy(
                    atab_hbm.at[aid2_v.at[q]],
                    art_v.at[pl.ds(qc * 128, 128)], sem_tab))
                tab_cps.append(pltpu.async_copy(
                    btab_hbm.at[bid2_v.at[q]],
                    alb_v.at[pl.ds(qc * 128, 128)], sem_tab))
            for c in audio_cps:
                c.wait()
            pltpu.sync_copy(audio_v, audio_out.at[pl.ds(cbase, CH)])
            for c in tab_cps:
                c.wait()
            pltpu.sync_copy(art_v, art_out.at[pl.ds(cbase, CH)])
            pltpu.sync_copy(alb_v, alb_out.at[pl.ds(cbase, CH)])
        pltpu.sync_copy(aid_v, aid_out.at[wid])
        pltpu.sync_copy(bid_v, bid_out.at[wid])

    return k(nodes3, nodesh3, audio2, aid_tab, bid_tab, art2, alb2)


BLK = 2048


def _tc_combine(audioP, artP, albP, sa, ra, rb, b2):
    def body(a_ref, r_ref, l_ref, sa_ref, ra_ref, rb_ref, b_ref, o_ref):
        a2 = a_ref[...]
        r2 = r_ref[...]
        l2 = l_ref[...]
        sa_ = sa_ref[...]
        ra_ = ra_ref[...]
        rb_ = rb_ref[...]
        a = a2[:, :D] + sa_ * (a2[:, D:] - a2[:, :D])
        r = r2[:, :D] + ra_ * (r2[:, D:] - r2[:, :D])
        l = l2[:, :D] + rb_ * (l2[:, D:] - l2[:, :D])
        y = a + r + l + b_ref[...]
        s = jnp.sum(y * y, axis=-1, keepdims=True)
        n = jnp.sqrt(s)
        o_ref[...] = y / jnp.maximum(n, 1e-12)

    return pl.pallas_call(
        body,
        grid=(B // BLK,),
        in_specs=[
            pl.BlockSpec((BLK, 128), lambda i: (i, 0)),
            pl.BlockSpec((BLK, 128), lambda i: (i, 0)),
            pl.BlockSpec((BLK, 128), lambda i: (i, 0)),
            pl.BlockSpec((BLK, 1), lambda i: (i, 0)),
            pl.BlockSpec((BLK, 1), lambda i: (i, 0)),
            pl.BlockSpec((BLK, 1), lambda i: (i, 0)),
            pl.BlockSpec((1, D), lambda i: (0, 0)),
        ],
        out_specs=pl.BlockSpec((BLK, D), lambda i: (i, 0)),
        out_shape=jax.ShapeDtypeStruct((B, D), jnp.float32),
    )(audioP, artP, albP, sa, ra, rb, b2)


def kernel(item_nodes, item_audio_emb, artist_ids, album_ids,
           artist_table, album_table, W, b):
    nodes = item_nodes.astype(jnp.int32)
    wa = W[:D] * jnp.float32(AUDIO_SCALE)
    wm = W[D:] * jnp.float32(METADATA_SCALE)
    artP = _tp_project(artist_table.T, wm)
    albP = _tp_project(album_table.T, wm)
    audP = _tp_project(item_audio_emb.T, wa)
    audio2 = audP.reshape(audP.shape[0] // 2, 128)
    art2 = artP.reshape(artP.shape[0] // 2, 128)
    alb2 = albP.reshape(albP.shape[0] // 2, 128)
    nodes3 = nodes.reshape(NW, NQ, 128)
    nodesh3 = (nodes >> 1).reshape(NW, NQ, 128)
    audioG, artG, albG, aidO, bidO = _sc_gather(
        nodes3, nodesh3, audio2,
        artist_ids.astype(jnp.int32), album_ids.astype(jnp.int32),
        art2, alb2)
    sa = (nodes & 1).astype(jnp.float32).reshape(B, 1)
    ra = (aidO & 1).astype(jnp.float32).reshape(B, 1)
    rb = (bidO & 1).astype(jnp.float32).reshape(B, 1)
    return _tc_combine(audioG, artG, albG, sa, ra, rb, b.reshape(1, D))


# speedup vs baseline: 1.3381x; 1.3381x over previous
import functools

import jax
import jax.numpy as jnp
from jax import lax
from jax.experimental import pallas as pl
from jax.experimental.pallas import tpu as pltpu
from jax.experimental.pallas import tpu_sc as plsc

B = 16384
D = 64
AUDIO_SCALE = 1.0
METADATA_SCALE = 1.0

_info = plsc.get_sparse_core_info()
NC, NS = _info.num_cores, _info.num_subcores
NW = NC * NS
BPW = B // NW
NQ = BPW // 128
CH = 256
NCHK = BPW // CH
QC = CH // 128

CB = 16384


def _tp_project(tableT, w):
    n = tableT.shape[1]
    grid = (n + CB - 1) // CB

    def body(t_ref, w_ref, o_ref):
        o_ref[...] = lax.dot_general(
            t_ref[...], w_ref[...], (((0,), (0,)), ((), ())),
            preferred_element_type=jnp.float32)

    return pl.pallas_call(
        body,
        grid=(grid,),
        in_specs=[
            pl.BlockSpec((D, CB), lambda i: (0, i)),
            pl.BlockSpec((D, D), lambda i: (0, 0)),
        ],
        out_specs=pl.BlockSpec((CB, D), lambda i: (i, 0)),
        out_shape=jax.ShapeDtypeStruct((n, D), jnp.float32),
    )(tableT, w)


def _sc_gather(nodes3, nodesh3, audio2, aid_tab, bid_tab, art2, alb2):
    mesh = plsc.VectorSubcoreMesh(core_axis_name="c", subcore_axis_name="s")

    @functools.partial(
        pl.kernel,
        mesh=mesh,
        out_type=(
            jax.ShapeDtypeStruct((B, 128), jnp.float32),
            jax.ShapeDtypeStruct((B, 128), jnp.float32),
            jax.ShapeDtypeStruct((B, 128), jnp.float32),
            jax.ShapeDtypeStruct((NW, NQ, 128), jnp.int32),
            jax.ShapeDtypeStruct((NW, NQ, 128), jnp.int32),
        ),
        scratch_types=[
            pltpu.VMEM((NQ, 128), jnp.int32),
            pltpu.VMEM((NQ, 128), jnp.int32),
            pltpu.VMEM((NQ, 128), jnp.int32),
            pltpu.VMEM((NQ, 128), jnp.int32),
            pltpu.VMEM((NQ, 128), jnp.int32),
            pltpu.VMEM((NQ, 128), jnp.int32),
            pltpu.VMEM((CH, 128), jnp.float32),
            pltpu.VMEM((CH, 128), jnp.float32),
            pltpu.VMEM((CH, 128), jnp.float32),
            pltpu.SemaphoreType.DMA,
            pltpu.SemaphoreType.DMA,
            pltpu.SemaphoreType.DMA,
        ],
    )
    def k(nodes_hbm, nodesh_hbm, audio_hbm, aid_hbm, bid_hbm, atab_hbm, btab_hbm,
          audio_out, art_out, alb_out, aid_out, bid_out,
          idx_v, idxh_v, aid_v, bid_v, aid2_v, bid2_v,
          audio_v, art_v, alb_v, sem_ids, sem_audio, sem_tab):
        wid = lax.axis_index("s") * NC + lax.axis_index("c")
        pltpu.sync_copy(nodes_hbm.at[wid], idx_v)
        pltpu.sync_copy(nodesh_hbm.at[wid], idxh_v)
        for h in range(NCHK):
            cbase = wid * BPW + h * CH
            audio_cps = []
            id_cps = []
            for qc in range(QC):
                q = h * QC + qc
                audio_cps.append(pltpu.async_copy(
                    audio_hbm.at[idxh_v.at[q]],
                    audio_v.at[pl.ds(qc * 128, 128)], sem_audio))
                id_cps.append(pltpu.async_copy(
                    aid_hbm.at[idx_v.at[q]], aid_v.at[q], sem_ids))
                id_cps.append(pltpu.async_copy(
                    bid_hbm.at[idx_v.at[q]], bid_v.at[q], sem_ids))
            for c in id_cps:
                c.wait()
            for qc in range(QC):
                q = h * QC + qc
                for j in range(8):
                    s = pl.ds(j * 16, 16)
                    aid2_v[q, s] = lax.shift_right_logical(aid_v[q, s], 1)
                    bid2_v[q, s] = lax.shift_right_logical(bid_v[q, s], 1)
            tab_cps = []
            for qc in range(QC):
                q = h * QC + qc
                tab_cps.append(pltpu.async_copy(
                    atab_hbm.at[aid2_v.at[q]],
                    art_v.at[pl.ds(qc * 128, 128)], sem_tab))
                tab_cps.append(pltpu.async_copy(
                    btab_hbm.at[bid2_v.at[q]],
                    alb_v.at[pl.ds(qc * 128, 128)], sem_tab))
            for c in audio_cps:
                c.wait()
            pltpu.sync_copy(audio_v, audio_out.at[pl.ds(cbase, CH)])
            for c in tab_cps:
                c.wait()
            pltpu.sync_copy(art_v, art_out.at[pl.ds(cbase, CH)])
            pltpu.sync_copy(alb_v, alb_out.at[pl.ds(cbase, CH)])
        pltpu.sync_copy(aid_v, aid_out.at[wid])
        pltpu.sync_copy(bid_v, bid_out.at[wid])

    return k(nodes3, nodesh3, audio2, aid_tab, bid_tab, art2, alb2)


BLK = 2048


def _tc_combine(audioP, artP, albP, sa, ra, rb, b2):
    def body(a_ref, r_ref, l_ref, sa_ref, ra_ref, rb_ref, b_ref, o_ref):
        a2 = a_ref[...]
        r2 = r_ref[...]
        l2 = l_ref[...]
        sa_ = sa_ref[...]
        ra_ = ra_ref[...]
        rb_ = rb_ref[...]
        a = a2[:, :D] + sa_ * (a2[:, D:] - a2[:, :D])
        r = r2[:, :D] + ra_ * (r2[:, D:] - r2[:, :D])
        l = l2[:, :D] + rb_ * (l2[:, D:] - l2[:, :D])
        y = a + r + l + b_ref[...]
        s = jnp.sum(y * y, axis=-1, keepdims=True)
        n = jnp.sqrt(s)
        o_ref[...] = y / jnp.maximum(n, 1e-12)

    return pl.pallas_call(
        body,
        grid=(B // BLK,),
        in_specs=[
            pl.BlockSpec((BLK, 128), lambda i: (i, 0)),
            pl.BlockSpec((BLK, 128), lambda i: (i, 0)),
            pl.BlockSpec((BLK, 128), lambda i: (i, 0)),
            pl.BlockSpec((BLK, 1), lambda i: (i, 0)),
            pl.BlockSpec((BLK, 1), lambda i: (i, 0)),
            pl.BlockSpec((BLK, 1), lambda i: (i, 0)),
            pl.BlockSpec((1, D), lambda i: (0, 0)),
        ],
        out_specs=pl.BlockSpec((BLK, D), lambda i: (i, 0)),
        out_shape=jax.ShapeDtypeStruct((B, D), jnp.float32),
    )(audioP, artP, albP, sa, ra, rb, b2)


def kernel(item_nodes, item_audio_emb, artist_ids, album_ids,
           artist_table, album_table, W, b):
    nodes = item_nodes.astype(jnp.int32)
    wa = W[:D] * jnp.float32(AUDIO_SCALE)
    wm = W[D:] * jnp.float32(METADATA_SCALE)
    artP = _tp_project(artist_table.T, wm)
    albP = _tp_project(album_table.T, wm)
    audP = _tp_project(item_audio_emb.T, wa)
    audio2 = audP.reshape(audP.shape[0] // 2, 128)
    art2 = artP.reshape(artP.shape[0] // 2, 128)
    alb2 = albP.reshape(albP.shape[0] // 2, 128)
    nodes3 = nodes.reshape(NW, NQ, 128)
    nodesh3 = (nodes >> 1).reshape(NW, NQ, 128)
    audioG, artG, albG, aidO, bidO = _sc_gather(
        nodes3, nodesh3, audio2,
        artist_ids.astype(jnp.int32), album_ids.astype(jnp.int32),
        art2, alb2)
    sa = (nodes & 1).astype(jnp.float32).reshape(B, 1)
    ra = (aidO & 1).astype(jnp.float32).reshape(B, 1)
    rb = (bidO & 1).astype(jnp.float32).reshape(B, 1)
    return _tc_combine(audioG, artG, albG, sa, ra, rb, b.reshape(1, D))


# speedup vs baseline: 2.3311x; 1.7421x over previous
import functools

import jax
import jax.numpy as jnp
from jax import lax
from jax.experimental import pallas as pl
from jax.experimental.pallas import tpu as pltpu
from jax.experimental.pallas import tpu_sc as plsc

B = 16384
D = 64
H = D // 2
AUDIO_SCALE = 1.0
METADATA_SCALE = 1.0

_info = plsc.get_sparse_core_info()
NC, NS = _info.num_cores, _info.num_subcores
NW = NC * NS
BPW = B // NW
NQ = BPW // 128
CH = 256
NCHK = BPW // CH
QC = CH // 128

CB = 16384
LB = 14
CBQ = CB // 4
LQ = 12
MQ = CBQ - 1


def _tp_pack(tableT, w):
    n = tableT.shape[1]
    grid = (n + CB - 1) // CB

    def body(t_ref, w_ref, o_ref):
        y = lax.dot_general(t_ref[...], w_ref[...], (((0,), (0,)), ((), ())),
                            preferred_element_type=jnp.float32)
        for qi in range(4):
            qs = y[qi * CBQ:(qi + 1) * CBQ]
            e = qs[:, :H].astype(jnp.bfloat16).astype(jnp.float32)
            h = qs[:, H:].astype(jnp.bfloat16).astype(jnp.float32)
            lo = lax.bitcast_convert_type(e, jnp.uint32) >> 16
            hi = lax.bitcast_convert_type(h, jnp.uint32)
            o_ref[:, qi * H:(qi + 1) * H] = lo | hi

    return pl.pallas_call(
        body,
        grid=(grid,),
        in_specs=[
            pl.BlockSpec((D, CB), lambda i: (0, i)),
            pl.BlockSpec((D, D), lambda i: (0, 0)),
        ],
        out_specs=pl.BlockSpec((CBQ, 128), lambda i: (i, 0)),
        out_shape=jax.ShapeDtypeStruct((grid * CBQ, 128), jnp.uint32),
    )(tableT, w)


def _sc_gather(nodes3, nodesr3, audio_p, aid_tab, bid_tab, art_p, alb_p):
    mesh = plsc.VectorSubcoreMesh(core_axis_name="c", subcore_axis_name="s")

    @functools.partial(
        pl.kernel,
        mesh=mesh,
        out_type=(
            jax.ShapeDtypeStruct((B, 128), jnp.uint32),
            jax.ShapeDtypeStruct((B, 128), jnp.uint32),
            jax.ShapeDtypeStruct((B, 128), jnp.uint32),
            jax.ShapeDtypeStruct((NW, NQ, 128), jnp.int32),
            jax.ShapeDtypeStruct((NW, NQ, 128), jnp.int32),
        ),
        scratch_types=[
            pltpu.VMEM((NQ, 128), jnp.int32),
            pltpu.VMEM((NQ, 128), jnp.int32),
            pltpu.VMEM((NQ, 128), jnp.int32),
            pltpu.VMEM((NQ, 128), jnp.int32),
            pltpu.VMEM((NQ, 128), jnp.int32),
            pltpu.VMEM((NQ, 128), jnp.int32),
            pltpu.VMEM((CH, 128), jnp.uint32),
            pltpu.VMEM((CH, 128), jnp.uint32),
            pltpu.VMEM((CH, 128), jnp.uint32),
            pltpu.SemaphoreType.DMA,
            pltpu.SemaphoreType.DMA,
            pltpu.SemaphoreType.DMA,
        ],
    )
    def k(nodes_hbm, nodesr_hbm, audio_hbm, aid_hbm, bid_hbm, atab_hbm, btab_hbm,
          audio_out, art_out, alb_out, aid_out, bid_out,
          idx_v, idxr_v, aid_v, bid_v, aidr_v, bidr_v,
          audio_v, art_v, alb_v, sem_ids, sem_audio, sem_tab):
        wid = lax.axis_index("s") * NC + lax.axis_index("c")
        pltpu.sync_copy(nodes_hbm.at[wid], idx_v)
        pltpu.sync_copy(nodesr_hbm.at[wid], idxr_v)
        for h in range(NCHK):
            cbase = wid * BPW + h * CH
            audio_cps = []
            id_cps = []
            for qc in range(QC):
                q = h * QC + qc
                audio_cps.append(pltpu.async_copy(
                    audio_hbm.at[idxr_v.at[q]],
                    audio_v.at[pl.ds(qc * 128, 128)], sem_audio))
                id_cps.append(pltpu.async_copy(
                    aid_hbm.at[idx_v.at[q]], aid_v.at[q], sem_ids))
                id_cps.append(pltpu.async_copy(
                    bid_hbm.at[idx_v.at[q]], bid_v.at[q], sem_ids))
            for c in id_cps:
                c.wait()
            for qc in range(QC):
                q = h * QC + qc
                for j in range(8):
                    s = pl.ds(j * 16, 16)
                    a = aid_v[q, s]
                    aidr_v[q, s] = lax.shift_left(
                        lax.shift_right_logical(a, LB), LQ) | (a & MQ)
                    bb = bid_v[q, s]
                    bidr_v[q, s] = lax.shift_left(
                        lax.shift_right_logical(bb, LB), LQ) | (bb & MQ)
            tab_cps = []
            for qc in range(QC):
                q = h * QC + qc
                tab_cps.append(pltpu.async_copy(
                    atab_hbm.at[aidr_v.at[q]],
                    art_v.at[pl.ds(qc * 128, 128)], sem_tab))
                tab_cps.append(pltpu.async_copy(
                    btab_hbm.at[bidr_v.at[q]],
                    alb_v.at[pl.ds(qc * 128, 128)], sem_tab))
            for c in audio_cps:
                c.wait()
            pltpu.sync_copy(audio_v, audio_out.at[pl.ds(cbase, CH)])
            for c in tab_cps:
                c.wait()
            pltpu.sync_copy(art_v, art_out.at[pl.ds(cbase, CH)])
            pltpu.sync_copy(alb_v, alb_out.at[pl.ds(cbase, CH)])
        pltpu.sync_copy(aid_v, aid_out.at[wid])
        pltpu.sync_copy(bid_v, bid_out.at[wid])

    return k(nodes3, nodesr3, audio_p, aid_tab, bid_tab, art_p, alb_p)


BLK = 2048


def _unpack_select(g, idv):
    b0 = ((idv >> LQ) & 1) == 1
    b1 = ((idv >> (LQ + 1)) & 1) == 1
    q01 = jnp.where(b0, g[:, H:2 * H], g[:, :H])
    q23 = jnp.where(b0, g[:, 3 * H:], g[:, 2 * H:3 * H])
    q = jnp.where(b1, q23, q01)
    e = lax.bitcast_convert_type(q << 16, jnp.float32)
    o = lax.bitcast_convert_type(q & jnp.uint32(0xFFFF0000), jnp.float32)
    return jnp.concatenate([e, o], axis=1)


def _tc_combine(audioG, artG, albG, nid, aid1, bid1, b2):
    def body(a_ref, r_ref, l_ref, ni_ref, ai_ref, bi_ref, b_ref, o_ref):
        pa = _unpack_select(a_ref[...], ni_ref[...])
        pr = _unpack_select(r_ref[...], ai_ref[...])
        pb = _unpack_select(l_ref[...], bi_ref[...])
        y = pa + pr + pb + b_ref[...]
        s = jnp.sum(y * y, axis=-1, keepdims=True)
        n = jnp.sqrt(s)
        o_ref[...] = y / jnp.maximum(n, 1e-12)

    return pl.pallas_call(
        body,
        grid=(B // BLK,),
        in_specs=[
            pl.BlockSpec((BLK, 128), lambda i: (i, 0)),
            pl.BlockSpec((BLK, 128), lambda i: (i, 0)),
            pl.BlockSpec((BLK, 128), lambda i: (i, 0)),
            pl.BlockSpec((BLK, 1), lambda i: (i, 0)),
            pl.BlockSpec((BLK, 1), lambda i: (i, 0)),
            pl.BlockSpec((BLK, 1), lambda i: (i, 0)),
            pl.BlockSpec((1, D), lambda i: (0, 0)),
        ],
        out_specs=pl.BlockSpec((BLK, D), lambda i: (i, 0)),
        out_shape=jax.ShapeDtypeStruct((B, D), jnp.float32),
    )(audioG, artG, albG, nid, aid1, bid1, b2)


def _packed_row(r):
    return ((r >> LB) << LQ) | (r & MQ)


def kernel(item_nodes, item_audio_emb, artist_ids, album_ids,
           artist_table, album_table, W, b):
    nodes = item_nodes.astype(jnp.int32)
    wa = W[:D] * jnp.float32(AUDIO_SCALE)
    wm = W[D:] * jnp.float32(METADATA_SCALE)
    artP = _tp_pack(artist_table.T, wm)
    albP = _tp_pack(album_table.T, wm)
    audP = _tp_pack(item_audio_emb.T, wa)
    nodes3 = nodes.reshape(NW, NQ, 128)
    nodesr3 = _packed_row(nodes).reshape(NW, NQ, 128)
    audioG, artG, albG, aidO, bidO = _sc_gather(
        nodes3, nodesr3, audP,
        artist_ids.astype(jnp.int32), album_ids.astype(jnp.int32),
        artP, albP)
    nid = nodes.reshape(B, 1)
    aid1 = aidO.reshape(B, 1)
    bid1 = bidO.reshape(B, 1)
    return _tc_combine(audioG, artG, albG, nid, aid1, bid1, b.reshape(1, D))


# speedup vs baseline: 2.7291x; 1.1707x over previous
import functools

import jax
import jax.numpy as jnp
from jax import lax
from jax.experimental import pallas as pl
from jax.experimental.pallas import tpu as pltpu
from jax.experimental.pallas import tpu_sc as plsc

B = 16384
D = 64
H = D // 2
AUDIO_SCALE = 1.0
METADATA_SCALE = 1.0

_info = plsc.get_sparse_core_info()
NC, NS = _info.num_cores, _info.num_subcores
NW = NC * NS
BPW = B // NW
NQ = BPW // 128
CH = 256
NCHK = BPW // CH
QC = CH // 128

CB = 16384
LB = 14
CBQ = CB // 4
LQ = 12
MQ = CBQ - 1


def _tp_pack(tableT, w):
    n = tableT.shape[1]
    grid = (n + CB - 1) // CB

    def body(t_ref, w_ref, o_ref):
        y = lax.dot_general(t_ref[...].astype(jnp.bfloat16),
                            w_ref[...].astype(jnp.bfloat16),
                            (((0,), (0,)), ((), ())),
                            preferred_element_type=jnp.float32)
        for qi in range(4):
            qs = y[qi * CBQ:(qi + 1) * CBQ]
            lo = lax.bitcast_convert_type(
                qs[:, :H].astype(jnp.bfloat16), jnp.uint16).astype(jnp.uint32)
            hi = lax.bitcast_convert_type(
                qs[:, H:].astype(jnp.bfloat16), jnp.uint16).astype(jnp.uint32)
            o_ref[:, qi * H:(qi + 1) * H] = lo | (hi << 16)

    return pl.pallas_call(
        body,
        grid=(grid,),
        in_specs=[
            pl.BlockSpec((D, CB), lambda i: (0, i)),
            pl.BlockSpec((D, D), lambda i: (0, 0)),
        ],
        out_specs=pl.BlockSpec((CBQ, 128), lambda i: (i, 0)),
        out_shape=jax.ShapeDtypeStruct((grid * CBQ, 128), jnp.uint32),
    )(tableT, w)


def _sc_meta_gather(nodes3, aid_tab, bid_tab, art_p, alb_p):
    mesh = plsc.VectorSubcoreMesh(core_axis_name="c", subcore_axis_name="s")

    @functools.partial(
        pl.kernel,
        mesh=mesh,
        out_type=(
            jax.ShapeDtypeStruct((B, 128), jnp.uint32),
            jax.ShapeDtypeStruct((B, 128), jnp.uint32),
            jax.ShapeDtypeStruct((NW, NQ, 128), jnp.int32),
            jax.ShapeDtypeStruct((NW, NQ, 128), jnp.int32),
        ),
        scratch_types=[
            pltpu.VMEM((NQ, 128), jnp.int32),
            pltpu.VMEM((NQ, 128), jnp.int32),
            pltpu.VMEM((NQ, 128), jnp.int32),
            pltpu.VMEM((NQ, 128), jnp.int32),
            pltpu.VMEM((NQ, 128), jnp.int32),
            pltpu.VMEM((CH, 128), jnp.uint32),
            pltpu.VMEM((CH, 128), jnp.uint32),
            pltpu.SemaphoreType.DMA,
            pltpu.SemaphoreType.DMA,
        ],
    )
    def k(nodes_hbm, aid_hbm, bid_hbm, atab_hbm, btab_hbm,
          art_out, alb_out, aid_out, bid_out,
          idx_v, aid_v, bid_v, aidr_v, bidr_v,
          art_v, alb_v, sem_ids, sem_tab):
        wid = lax.axis_index("s") * NC + lax.axis_index("c")
        pltpu.sync_copy(nodes_hbm.at[wid], idx_v)
        id_cps = []
        for q in range(NQ):
            id_cps.append(pltpu.async_copy(
                aid_hbm.at[idx_v.at[q]], aid_v.at[q], sem_ids))
            id_cps.append(pltpu.async_copy(
                bid_hbm.at[idx_v.at[q]], bid_v.at[q], sem_ids))
        for c in id_cps:
            c.wait()
        for q in range(NQ):
            for j in range(8):
                s = pl.ds(j * 16, 16)
                a = aid_v[q, s]
                aidr_v[q, s] = lax.shift_left(
                    lax.shift_right_logical(a, LB), LQ) | (a & MQ)
                bb = bid_v[q, s]
                bidr_v[q, s] = lax.shift_left(
                    lax.shift_right_logical(bb, LB), LQ) | (bb & MQ)
        pltpu.sync_copy(aid_v, aid_out.at[wid])
        pltpu.sync_copy(bid_v, bid_out.at[wid])
        for h in range(NCHK):
            cbase = wid * BPW + h * CH
            tab_cps = []
            for qc in range(QC):
                q = h * QC + qc
                tab_cps.append(pltpu.async_copy(
                    atab_hbm.at[aidr_v.at[q]],
                    art_v.at[pl.ds(qc * 128, 128)], sem_tab))
                tab_cps.append(pltpu.async_copy(
                    btab_hbm.at[bidr_v.at[q]],
                    alb_v.at[pl.ds(qc * 128, 128)], sem_tab))
            for c in tab_cps:
                c.wait()
            pltpu.sync_copy(art_v, art_out.at[pl.ds(cbase, CH)])
            pltpu.sync_copy(alb_v, alb_out.at[pl.ds(cbase, CH)])

    return k(nodes3, aid_tab, bid_tab, art_p, alb_p)


def _sc_audio_gather(nodesr3, audio_p):
    mesh = plsc.VectorSubcoreMesh(core_axis_name="c", subcore_axis_name="s")

    @functools.partial(
        pl.kernel,
        mesh=mesh,
        out_type=jax.ShapeDtypeStruct((B, 128), jnp.uint32),
        scratch_types=[
            pltpu.VMEM((NQ, 128), jnp.int32),
            pltpu.VMEM((BPW, 128), jnp.uint32),
            pltpu.SemaphoreType.DMA,
        ],
    )
    def k(nodesr_hbm, audio_hbm, audio_out, idxr_v, audio_v, sem):
        wid = lax.axis_index("s") * NC + lax.axis_index("c")
        pltpu.sync_copy(nodesr_hbm.at[wid], idxr_v)
        cps = []
        for q in range(NQ):
            cps.append(pltpu.async_copy(
                audio_hbm.at[idxr_v.at[q]],
                audio_v.at[pl.ds(q * 128, 128)], sem))
        for c in cps:
            c.wait()
        pltpu.sync_copy(audio_v, audio_out.at[pl.ds(wid * BPW, BPW)])

    return k(nodesr3, audio_p)


BLK = 4096


def _unpack_select(g, idv):
    b0 = ((idv >> LQ) & 1) == 1
    b1 = ((idv >> (LQ + 1)) & 1) == 1
    q01 = jnp.where(b0, g[:, H:2 * H], g[:, :H])
    q23 = jnp.where(b0, g[:, 3 * H:], g[:, 2 * H:3 * H])
    q = jnp.where(b1, q23, q01)
    e = lax.bitcast_convert_type(q << 16, jnp.float32)
    o = lax.bitcast_convert_type(q & jnp.uint32(0xFFFF0000), jnp.float32)
    return jnp.concatenate([e, o], axis=1)


def _tc_combine(audioG, artG, albG, nid, aid1, bid1, b2):
    def body(a_ref, r_ref, l_ref, ni_ref, ai_ref, bi_ref, b_ref, o_ref):
        pa = _unpack_select(a_ref[...], ni_ref[...])
        pr = _unpack_select(r_ref[...], ai_ref[...])
        pb = _unpack_select(l_ref[...], bi_ref[...])
        y = pa + pr + pb + b_ref[...]
        s = jnp.sum(y * y, axis=-1, keepdims=True)
        n = jnp.sqrt(s)
        o_ref[...] = y / jnp.maximum(n, 1e-12)

    return pl.pallas_call(
        body,
        grid=(B // BLK,),
        in_specs=[
            pl.BlockSpec((BLK, 128), lambda i: (i, 0)),
            pl.BlockSpec((BLK, 128), lambda i: (i, 0)),
            pl.BlockSpec((BLK, 128), lambda i: (i, 0)),
            pl.BlockSpec((BLK, 1), lambda i: (i, 0)),
            pl.BlockSpec((BLK, 1), lambda i: (i, 0)),
            pl.BlockSpec((BLK, 1), lambda i: (i, 0)),
            pl.BlockSpec((1, D), lambda i: (0, 0)),
        ],
        out_specs=pl.BlockSpec((BLK, D), lambda i: (i, 0)),
        out_shape=jax.ShapeDtypeStruct((B, D), jnp.float32),
    )(audioG, artG, albG, nid, aid1, bid1, b2)


def _packed_row(r):
    return ((r >> LB) << LQ) | (r & MQ)


def kernel(item_nodes, item_audio_emb, artist_ids, album_ids,
           artist_table, album_table, W, b):
    nodes = item_nodes.astype(jnp.int32)
    wa = W[:D] * jnp.float32(AUDIO_SCALE)
    wm = W[D:] * jnp.float32(METADATA_SCALE)
    artP = _tp_pack(artist_table.T, wm)
    albP = _tp_pack(album_table.T, wm)
    nodes3 = nodes.reshape(NW, NQ, 128)
    nodesr3 = _packed_row(nodes).reshape(NW, NQ, 128)
    artG, albG, aidO, bidO = _sc_meta_gather(
        nodes3, artist_ids.astype(jnp.int32), album_ids.astype(jnp.int32),
        artP, albP)
    audP = _tp_pack(item_audio_emb.T, wa)
    audioG = _sc_audio_gather(nodesr3, audP)
    nid = nodes.reshape(B, 1)
    aid1 = aidO.reshape(B, 1)
    bid1 = bidO.reshape(B, 1)
    return _tc_combine(audioG, artG, albG, nid, aid1, bid1, b.reshape(1, D))
